# baseline (speedup 1.0000x reference)
import jax
import jax.numpy as jnp
from jax.experimental import pallas as pl
from jax.experimental.pallas import tpu as pltpu

B, D, L = 64, 576, 196
Cc, K = 10, 20
M = Cc * K
LB = 49
NSTEPS = L // LB
CPAD = 128


def _tm_kernel(lit_ref, mask_ref, alpha_ref, out_ref, acc_ref):
    i = pl.program_id(0)
    mask = mask_ref[...].astype(jnp.float32)
    x = lit_ref[...].reshape(LB * B, D).astype(jnp.bfloat16)
    s = jax.lax.dot_general(x, mask.astype(jnp.bfloat16),
                            (((1,), (1,)), ((), ())),
                            preferred_element_type=jnp.float32)
    m = jnp.max(s.reshape(LB, B, M), axis=0)

    @pl.when(i == 0)
    def _init():
        acc_ref[...] = m

    @pl.when(i > 0)
    def _acc():
        acc_ref[...] = jnp.maximum(acc_ref[...], m)

    @pl.when(i == NSTEPS - 1)
    def _finish():
        count = jnp.sum(mask, axis=1)
        clause = (acc_ref[...] >= count[None, :] - 0.5).astype(jnp.float32)
        weighted = clause * alpha_ref[...]
        m_idx = jax.lax.broadcasted_iota(jnp.int32, (Cc, M), 1)
        c_idx = jax.lax.broadcasted_iota(jnp.int32, (Cc, M), 0)
        sign = jnp.where((m_idx % K) < (K // 2), 1.0, -1.0)
        voteT = jnp.where(m_idx // K == c_idx, sign, 0.0)
        out_ref[...] = jax.lax.dot_general(
            voteT, weighted, (((1,), (1,)), ((), ())),
            preferred_element_type=jnp.float32)


def kernel(literals, clause_mask, alpha):
    lit_t = literals.transpose(2, 0, 1)
    alpha2 = alpha.reshape(1, M).astype(jnp.float32)

    out = pl.pallas_call(
        _tm_kernel,
        grid=(NSTEPS,),
        in_specs=[
            pl.BlockSpec((LB, B, D), lambda i: (i, 0, 0)),
            pl.BlockSpec((M, D), lambda i: (0, 0)),
            pl.BlockSpec((1, M), lambda i: (0, 0)),
        ],
        out_specs=pl.BlockSpec((Cc, B), lambda i: (0, 0)),
        out_shape=jax.ShapeDtypeStruct((Cc, B), jnp.float32),
        scratch_shapes=[pltpu.VMEM((B, M), jnp.float32)],
        compiler_params=pltpu.CompilerParams(
            dimension_semantics=("arbitrary",),
        ),
    )(lit_t, clause_mask, alpha2)
    return out.T


# speedup vs baseline: 1.0159x; 1.0159x over previous
import jax
import jax.numpy as jnp
from jax.experimental import pallas as pl
from jax.experimental.pallas import tpu as pltpu

B, D, L = 64, 576, 196
Cc, K = 10, 20
M = Cc * K
LB = 49
NSTEPS = L // LB


def _tm_kernel(lit_ref, mask_ref, alpha_ref, out_ref, acc_ref):
    i = pl.program_id(0)
    mask_b = mask_ref[...].astype(jnp.bfloat16)
    x = lit_ref[...].reshape(LB * B, D).astype(jnp.bfloat16)
    s = jax.lax.dot_general(x, mask_b,
                            (((1,), (1,)), ((), ())),
                            preferred_element_type=jnp.float32)
    m = jnp.max(s.reshape(LB, B, M), axis=0)

    @pl.when(i == 0)
    def _init():
        acc_ref[...] = m

    @pl.when(i > 0)
    def _acc():
        acc_ref[...] = jnp.maximum(acc_ref[...], m)

    @pl.when(i == NSTEPS - 1)
    def _finish():
        count = jnp.sum(mask_ref[...].astype(jnp.float32), axis=1)
        clause = (acc_ref[...] >= count[None, :] - 0.5).astype(jnp.float32)
        weighted = clause * alpha_ref[...]
        m_idx = jax.lax.broadcasted_iota(jnp.int32, (Cc, M), 1)
        c_idx = jax.lax.broadcasted_iota(jnp.int32, (Cc, M), 0)
        sign = jnp.where((m_idx % K) < (K // 2), 1.0, -1.0)
        voteT = jnp.where(m_idx // K == c_idx, sign, 0.0)
        out_ref[...] = jax.lax.dot_general(
            voteT, weighted, (((1,), (1,)), ((), ())),
            preferred_element_type=jnp.float32)


def kernel(literals, clause_mask, alpha):
    lit_t = literals.transpose(2, 0, 1)
    alpha2 = alpha.reshape(1, M).astype(jnp.float32)

    out = pl.pallas_call(
        _tm_kernel,
        grid=(NSTEPS,),
        in_specs=[
            pl.BlockSpec((LB, B, D), lambda i: (i, 0, 0)),
            pl.BlockSpec((M, D), lambda i: (0, 0)),
            pl.BlockSpec((1, M), lambda i: (0, 0)),
        ],
        out_specs=pl.BlockSpec((Cc, B), lambda i: (0, 0)),
        out_shape=jax.ShapeDtypeStruct((Cc, B), jnp.float32),
        scratch_shapes=[pltpu.VMEM((B, M), jnp.float32)],
        compiler_params=pltpu.CompilerParams(
            dimension_semantics=("arbitrary",),
        ),
    )(lit_t, clause_mask, alpha2)
    return out.T
